# Initial kernel scaffold; baseline (speedup 1.0000x reference)
import jax
import jax.numpy as jnp
from jax.experimental import pallas as pl


def kernel(x, gn1_s, gn1_b, conv1_w, conv1_b, gn2_s, gn2_b, conv2_w, conv2_b, router_w, eg_w, eg_b, eu_w, eu_b, ed_w, ed_b, sg_w, sg_b, su_w, su_b, sd_w, sd_b):
    raise NotImplementedError("write your pallas kernel here")


# speedup vs baseline: 1.4936x; 1.4936x over previous
import functools

import jax
import jax.numpy as jnp
from jax.experimental import pallas as pl

B = 4
C = 384
H = 24
W = 24
HW = H * W
N = B * HW
E = 8
F = 768
GROUPS = 32
CPG = C // GROUPS
EPS = 1e-6


def _group_stats(x, gmat):
    s = jnp.sum(x, axis=0, keepdims=True)
    sq = jnp.sum(x * x, axis=0, keepdims=True)
    denom = float(CPG * HW)
    mean = jnp.dot(s, gmat, preferred_element_type=jnp.float32) / denom
    ex2 = jnp.dot(sq, gmat, preferred_element_type=jnp.float32) / denom
    var = ex2 - mean * mean
    return mean, var


def _gn_silu(x, gmat, scale, bias):
    mean, var = _group_stats(x, gmat)
    xh = (x - mean) * jax.lax.rsqrt(var + EPS) * scale + bias
    return xh * jax.lax.logistic(xh)


def _conv3x3(a_bf16, w_ref):
    a3 = jnp.pad(a_bf16.reshape(H, W, C), ((1, 1), (1, 1), (0, 0)))
    acc = jnp.zeros((HW, C), jnp.float32)
    for k in range(9):
        dy, dx = k // 3, k % 3
        win = a3[dy:dy + H, dx:dx + W].reshape(HW, C)
        acc = acc + jnp.dot(win, w_ref[k], preferred_element_type=jnp.float32)
    return acc


def _resnet_body(x_ref, w1_ref, w2_ref, gn1s_ref, gn1b_ref, c1b_ref,
                 gn2s_ref, gn2b_ref, c2b_ref, rw_ref,
                 r_ref, t_ref, comb_ref):
    x = x_ref[0]
    ii = jax.lax.broadcasted_iota(jnp.int32, (C, C), 0) // CPG
    jj = jax.lax.broadcasted_iota(jnp.int32, (C, C), 1) // CPG
    gmat = (ii == jj).astype(jnp.float32)

    a1 = _gn_silu(x, gmat, gn1s_ref[...], gn1b_ref[...]).astype(jnp.bfloat16)
    h1 = _conv3x3(a1, w1_ref) + c1b_ref[...]
    a2 = _gn_silu(h1, gmat, gn2s_ref[...], gn2b_ref[...]).astype(jnp.bfloat16)
    h2 = _conv3x3(a2, w2_ref) + c2b_ref[...]
    r = x + h2

    logits = jnp.dot(r, rw_ref[...], preferred_element_type=jnp.float32)
    m = jnp.max(logits, axis=1, keepdims=True)
    ex = jnp.exp(logits - m)
    probs = ex / jnp.sum(ex, axis=1, keepdims=True)

    lane = jax.lax.broadcasted_iota(jnp.int32, (HW, E), 1)
    v1 = jnp.max(probs, axis=1, keepdims=True)
    i1 = jnp.min(jnp.where(probs == v1, lane, E), axis=1, keepdims=True)
    p2 = jnp.where(lane == i1, -jnp.inf, probs)
    v2 = jnp.max(p2, axis=1, keepdims=True)
    i2 = jnp.min(jnp.where(p2 == v2, lane, E), axis=1, keepdims=True)
    s = v1 + v2
    comb = (jnp.where(lane == i1, v1 / s, 0.0)
            + jnp.where(lane == i2, v2 / s, 0.0))

    r_ref[0] = r
    t_ref[0] = r.astype(jnp.bfloat16)
    comb_ref[0] = comb


def _gelu_tanh(g):
    c = 0.7978845608028654
    return 0.5 * g * (1.0 + jnp.tanh(c * (g + 0.044715 * g * g * g)))


def _expert_body(t_ref, r_ref, comb_ref, egw_ref, euw_ref, edw_ref,
                 egb_ref, eub_ref, edb_ref, out_ref):
    e = pl.program_id(0)
    t = t_ref[...]
    g = jnp.dot(t, egw_ref[0], preferred_element_type=jnp.float32) + egb_ref[0]
    u = jnp.dot(t, euw_ref[0], preferred_element_type=jnp.float32) + eub_ref[0]
    hh = (_gelu_tanh(g) * u).astype(jnp.bfloat16)
    o = jnp.dot(hh, edw_ref[0], preferred_element_type=jnp.float32) + edb_ref[0]
    lane = jax.lax.broadcasted_iota(jnp.int32, (N, E + 1), 1)
    c = jnp.sum(jnp.where(lane == e, comb_ref[...], 0.0), axis=1, keepdims=True)
    contrib = o * c

    @pl.when(e == 0)
    def _init():
        out_ref[...] = r_ref[...] + contrib

    @pl.when(e != 0)
    def _acc():
        out_ref[...] = out_ref[...] + contrib


@jax.jit
def kernel(x, gn1_s, gn1_b, conv1_w, conv1_b, gn2_s, gn2_b, conv2_w, conv2_b,
           router_w, eg_w, eg_b, eu_w, eu_b, ed_w, ed_b,
           sg_w, sg_b, su_w, su_b, sd_w, sd_b):
    f32 = jnp.float32
    bf16 = jnp.bfloat16
    xt = x.transpose(0, 2, 3, 1).reshape(B, HW, C)
    w1m = conv1_w.transpose(2, 3, 1, 0).reshape(9, C, C).astype(bf16)
    w2m = conv2_w.transpose(2, 3, 1, 0).reshape(9, C, C).astype(bf16)

    resnet = pl.pallas_call(
        _resnet_body,
        grid=(B,),
        in_specs=[
            pl.BlockSpec((1, HW, C), lambda b: (b, 0, 0)),
            pl.BlockSpec((9, C, C), lambda b: (0, 0, 0)),
            pl.BlockSpec((9, C, C), lambda b: (0, 0, 0)),
            pl.BlockSpec((1, C), lambda b: (0, 0)),
            pl.BlockSpec((1, C), lambda b: (0, 0)),
            pl.BlockSpec((1, C), lambda b: (0, 0)),
            pl.BlockSpec((1, C), lambda b: (0, 0)),
            pl.BlockSpec((1, C), lambda b: (0, 0)),
            pl.BlockSpec((1, C), lambda b: (0, 0)),
            pl.BlockSpec((C, E), lambda b: (0, 0)),
        ],
        out_specs=[
            pl.BlockSpec((1, HW, C), lambda b: (b, 0, 0)),
            pl.BlockSpec((1, HW, C), lambda b: (b, 0, 0)),
            pl.BlockSpec((1, HW, E), lambda b: (b, 0, 0)),
        ],
        out_shape=[
            jax.ShapeDtypeStruct((B, HW, C), f32),
            jax.ShapeDtypeStruct((B, HW, C), bf16),
            jax.ShapeDtypeStruct((B, HW, E), f32),
        ],
    )
    r, t, comb = resnet(
        xt, w1m, w2m,
        gn1_s.reshape(1, C), gn1_b.reshape(1, C), conv1_b.reshape(1, C),
        gn2_s.reshape(1, C), gn2_b.reshape(1, C), conv2_b.reshape(1, C),
        router_w,
    )
    r = r.reshape(N, C)
    t = t.reshape(N, C)
    comb9 = jnp.concatenate(
        [comb.reshape(N, E), jnp.ones((N, 1), f32)], axis=1)

    egw = jnp.concatenate([eg_w, sg_w[None]], axis=0).astype(bf16)
    euw = jnp.concatenate([eu_w, su_w[None]], axis=0).astype(bf16)
    edw = jnp.concatenate([ed_w, sd_w[None]], axis=0).astype(bf16)
    egb = jnp.concatenate([eg_b, sg_b[None]], axis=0).reshape(E + 1, 1, F)
    eub = jnp.concatenate([eu_b, su_b[None]], axis=0).reshape(E + 1, 1, F)
    edb = jnp.concatenate([ed_b, sd_b[None]], axis=0).reshape(E + 1, 1, C)

    moe = pl.pallas_call(
        _expert_body,
        grid=(E + 1,),
        in_specs=[
            pl.BlockSpec((N, C), lambda e: (0, 0)),
            pl.BlockSpec((N, C), lambda e: (0, 0)),
            pl.BlockSpec((N, E + 1), lambda e: (0, 0)),
            pl.BlockSpec((1, C, F), lambda e: (e, 0, 0)),
            pl.BlockSpec((1, C, F), lambda e: (e, 0, 0)),
            pl.BlockSpec((1, F, C), lambda e: (e, 0, 0)),
            pl.BlockSpec((1, 1, F), lambda e: (e, 0, 0)),
            pl.BlockSpec((1, 1, F), lambda e: (e, 0, 0)),
            pl.BlockSpec((1, 1, C), lambda e: (e, 0, 0)),
        ],
        out_specs=pl.BlockSpec((N, C), lambda e: (0, 0)),
        out_shape=jax.ShapeDtypeStruct((N, C), f32),
    )
    out = moe(t, r, comb9, egw, euw, edw, egb, eub, edb)
    return out.reshape(B, H, W, C).transpose(0, 3, 1, 2)
